# baseline (speedup 1.0000x reference)
import dataclasses

import jax
import jax.numpy as jnp
from jax.experimental import pallas as pl
from jax.experimental.pallas import tpu as pltpu
from jax.experimental.pallas import tpu_sc as plsc

B, T, D = 64, 576, 768
E, R = 8, 192
RB = 8
RSTEPS = B // RB
SB = 4
ASTEPS = B // SB
LANES = 16
CHUNKS = B // LANES

_SC_COMPILER_PARAMS = pltpu.CompilerParams()
if "needs_layout_passes" in pltpu.CompilerParams.__dataclass_fields__:
    _SC_COMPILER_PARAMS = dataclasses.replace(
        _SC_COMPILER_PARAMS, needs_layout_passes=False)


def _routing_kernel(tokens_ref, gate_W_ref, gate_b_ref,
                    logits_ref, sel_ref, top1_ref, pt_ref,
                    pooled_ref):
    i = pl.program_id(0)
    pooled_ref[i] = jnp.mean(tokens_ref[...], axis=1)

    @pl.when(i == RSTEPS - 1)
    def _finish():
        pooled = pooled_ref[...].reshape(B, D)
        logits = jnp.dot(pooled, gate_W_ref[...],
                         preferred_element_type=jnp.float32) + gate_b_ref[...]
        m = jnp.max(logits, axis=-1, keepdims=True)
        p = jnp.exp(logits - m)
        p = p / jnp.sum(p, axis=-1, keepdims=True)
        top1 = jnp.max(p, axis=-1, keepdims=True)
        iota_e = jax.lax.broadcasted_iota(jnp.int32, (B, E), 1)
        sel = jnp.min(jnp.where(p == top1, iota_e, E), axis=-1, keepdims=True)
        logits_ref[...] = logits
        sel_ref[...] = sel
        top1_ref[...] = top1
        pt_ref[...] = p.T


def _sc_gating_kernel(pt_hbm, ew_hbm, imp_hbm, load_hbm,
                      pt_v, ew_v, imp_v, load_v, sem):
    c = jax.lax.axis_index("c")
    s = jax.lax.axis_index("s")

    @pl.when(jnp.logical_and(c == 0, s == 0))
    def _():
        pltpu.async_copy(pt_hbm, pt_v, sem).wait()
        zero16 = jnp.zeros((LANES,), jnp.float32)
        for k in range(B * E // LANES):
            ew_v[pl.ds(k * LANES, LANES)] = zero16
        imp_acc = [jnp.float32(0.0)] * E
        load_acc = [jnp.float32(0.0)] * E
        for chunk in range(CHUNKS):
            base = chunk * LANES
            vecs = [pt_v[e, pl.ds(base, LANES)] for e in range(E)]
            maxp = vecs[0]
            for e in range(1, E):
                maxp = jnp.maximum(maxp, vecs[e])
            selv = jnp.full((LANES,), E - 1, jnp.int32)
            for e in range(E - 2, -1, -1):
                selv = jnp.where(vecs[e] == maxp,
                                 jnp.full((LANES,), e, jnp.int32), selv)
            idx = (jax.lax.iota(jnp.int32, LANES) + base) * E + selv
            plsc.store_scatter(ew_v, [idx], maxp)
            for e in range(E):
                mask = selv == e
                imp_acc[e] += jnp.sum(jnp.where(mask, maxp, 0.0))
                load_acc[e] += jnp.sum(jnp.where(mask, 1.0 / B, 0.0))
        lane = jax.lax.iota(jnp.int32, LANES)
        impv = zero16
        loadv = zero16
        for e in range(E):
            impv = jnp.where(lane == e, imp_acc[e], impv)
            loadv = jnp.where(lane == e, load_acc[e], loadv)
        imp_v[...] = impv
        load_v[...] = loadv
        pltpu.async_copy(ew_v, ew_hbm, sem).wait()
        pltpu.async_copy(imp_v, imp_hbm, sem).wait()
        pltpu.async_copy(load_v, load_hbm, sem).wait()


def _adapter_kernel(sel_sp, t1_sp, tokens_ref, wd_ref, wu_ref, bd_ref, bu_ref,
                    out_ref):
    g = pl.program_id(0)
    for j in range(SB):
        e = sel_sp[g * SB + j]
        x = tokens_ref[j]
        h = jnp.dot(x.astype(jnp.bfloat16), wd_ref[e],
                    preferred_element_type=jnp.float32) + bd_ref[e, :][None, :]
        h = jax.nn.gelu(h)
        y = jnp.dot(h.astype(jnp.bfloat16), wu_ref[e],
                    preferred_element_type=jnp.float32) + bu_ref[e, :][None, :]
        out_ref[j] = (x + y) * t1_sp[g * SB + j]


@jax.jit
def kernel(tokens, spatial_shape, gate_W, gate_b, W_down, b_down, W_up, b_up):
    del spatial_shape
    logits, sel, top1, pt = pl.pallas_call(
        _routing_kernel,
        grid=(RSTEPS,),
        in_specs=[
            pl.BlockSpec((RB, T, D), lambda i: (i, 0, 0)),
            pl.BlockSpec((D, E), lambda i: (0, 0)),
            pl.BlockSpec((1, E), lambda i: (0, 0)),
        ],
        out_specs=[
            pl.BlockSpec((B, E), lambda i: (0, 0)),
            pl.BlockSpec((B, 1), lambda i: (0, 0)),
            pl.BlockSpec((B, 1), lambda i: (0, 0)),
            pl.BlockSpec((E, B), lambda i: (0, 0)),
        ],
        out_shape=[
            jax.ShapeDtypeStruct((B, E), jnp.float32),
            jax.ShapeDtypeStruct((B, 1), jnp.int32),
            jax.ShapeDtypeStruct((B, 1), jnp.float32),
            jax.ShapeDtypeStruct((E, B), jnp.float32),
        ],
        scratch_shapes=[pltpu.VMEM((RSTEPS, RB, D), jnp.float32)],
        compiler_params=pltpu.CompilerParams(
            dimension_semantics=("arbitrary",)),
    )(tokens, gate_W, gate_b.reshape(1, E))

    sc_gating = pl.kernel(
        _sc_gating_kernel,
        out_type=[
            jax.ShapeDtypeStruct((B * E,), jnp.float32),
            jax.ShapeDtypeStruct((LANES,), jnp.float32),
            jax.ShapeDtypeStruct((LANES,), jnp.float32),
        ],
        mesh=plsc.VectorSubcoreMesh(core_axis_name="c", subcore_axis_name="s"),
        compiler_params=_SC_COMPILER_PARAMS,
        scratch_types=[
            pltpu.VMEM((E, B), jnp.float32),
            pltpu.VMEM((B * E,), jnp.float32),
            pltpu.VMEM((LANES,), jnp.float32),
            pltpu.VMEM((LANES,), jnp.float32),
            pltpu.SemaphoreType.DMA,
        ],
    )
    ew_flat, imp16, load16 = sc_gating(pt)

    grid_spec = pltpu.PrefetchScalarGridSpec(
        num_scalar_prefetch=2,
        grid=(ASTEPS,),
        in_specs=[
            pl.BlockSpec((SB, T, D), lambda g, sp, tp: (g, 0, 0)),
            pl.BlockSpec((E, D, R), lambda g, sp, tp: (0, 0, 0)),
            pl.BlockSpec((E, R, D), lambda g, sp, tp: (0, 0, 0)),
            pl.BlockSpec((E, R), lambda g, sp, tp: (0, 0)),
            pl.BlockSpec((E, D), lambda g, sp, tp: (0, 0)),
        ],
        out_specs=pl.BlockSpec((SB, T, D), lambda g, sp, tp: (g, 0, 0)),
    )
    weighted = pl.pallas_call(
        _adapter_kernel,
        grid_spec=grid_spec,
        out_shape=jax.ShapeDtypeStruct((B, T, D), jnp.float32),
        compiler_params=pltpu.CompilerParams(
            dimension_semantics=("arbitrary",)),
    )(sel.reshape(B), top1.reshape(B), tokens, W_down.astype(jnp.bfloat16),
      W_up.astype(jnp.bfloat16), b_down, b_up)

    return (weighted, logits, sel, ew_flat.reshape(B, E), imp16[:E],
            load16[:E])


# speedup vs baseline: 1.0265x; 1.0265x over previous
import dataclasses

import jax
import jax.numpy as jnp
from jax.experimental import pallas as pl
from jax.experimental.pallas import tpu as pltpu
from jax.experimental.pallas import tpu_sc as plsc

B, T, D = 64, 576, 768
E, R = 8, 192
RB = 4
RSTEPS = B // RB
SB = 4
ASTEPS = B // SB
LANES = 16
CHUNKS = B // LANES

_SC_COMPILER_PARAMS = pltpu.CompilerParams()
if "needs_layout_passes" in pltpu.CompilerParams.__dataclass_fields__:
    _SC_COMPILER_PARAMS = dataclasses.replace(
        _SC_COMPILER_PARAMS, needs_layout_passes=False)


def _moe_kernel(tokens_r_ref, tokens_a_ref, gate_W_ref, gate_b_ref,
                wd_ref, wu_ref, bd_ref, bu_ref,
                out_ref, logits_ref, sel_ref, selt_out_ref, t1t_out_ref,
                pooled_ref, sel_smem, t1_smem, sem):
    i = pl.program_id(0)

    @pl.when(i < RSTEPS)
    def _route():
        pooled_ref[i] = jnp.mean(tokens_r_ref[...], axis=1)

    @pl.when(i == RSTEPS - 1)
    def _finish_route():
        pooled = pooled_ref[...].reshape(B, D)
        logits = jnp.dot(pooled, gate_W_ref[...],
                         preferred_element_type=jnp.float32) + gate_b_ref[...]
        m = jnp.max(logits, axis=-1, keepdims=True)
        p = jnp.exp(logits - m)
        p = p / jnp.sum(p, axis=-1, keepdims=True)
        top1 = jnp.max(p, axis=-1, keepdims=True)
        iota_e = jax.lax.broadcasted_iota(jnp.int32, (B, E), 1)
        sel = jnp.min(jnp.where(p == top1, iota_e, E), axis=-1, keepdims=True)
        logits_ref[...] = logits
        sel_ref[...] = sel
        selt_out_ref[...] = sel.reshape(1, B)
        t1t_out_ref[...] = top1.reshape(1, B)
        cp1 = pltpu.make_async_copy(selt_out_ref, sel_smem, sem)
        cp1.start()
        cp1.wait()
        cp2 = pltpu.make_async_copy(t1t_out_ref, t1_smem, sem)
        cp2.start()
        cp2.wait()

    @pl.when(i >= RSTEPS)
    def _adapt():
        g = i - RSTEPS
        for j in range(SB):
            e = sel_smem[0, g * SB + j]
            t1 = t1_smem[0, g * SB + j]
            x = tokens_a_ref[j]
            h = jnp.dot(x.astype(jnp.bfloat16), wd_ref[e],
                        preferred_element_type=jnp.float32) + bd_ref[e, :][None, :]
            h = jax.nn.gelu(h)
            y = jnp.dot(h.astype(jnp.bfloat16), wu_ref[e],
                        preferred_element_type=jnp.float32) + bu_ref[e, :][None, :]
            out_ref[j] = (x + y) * t1


def _sc_gating_kernel(selt_hbm, t1t_hbm, ew_hbm, imp_hbm, load_hbm,
                      selt_v, t1t_v, ew_v, imp_v, load_v, sem):
    c = jax.lax.axis_index("c")
    s = jax.lax.axis_index("s")

    @pl.when(jnp.logical_and(c == 0, s == 0))
    def _():
        pltpu.async_copy(selt_hbm, selt_v, sem).wait()
        pltpu.async_copy(t1t_hbm, t1t_v, sem).wait()
        zero16 = jnp.zeros((LANES,), jnp.float32)
        for k in range(B * E // LANES):
            ew_v[pl.ds(k * LANES, LANES)] = zero16
        imp_acc = [jnp.float32(0.0)] * E
        load_acc = [jnp.float32(0.0)] * E
        for chunk in range(CHUNKS):
            base = chunk * LANES
            selv = selt_v[0, pl.ds(base, LANES)]
            t1v = t1t_v[0, pl.ds(base, LANES)]
            idx = (jax.lax.iota(jnp.int32, LANES) + base) * E + selv
            plsc.store_scatter(ew_v, [idx], t1v)
            for e in range(E):
                mask = selv == e
                imp_acc[e] += jnp.sum(jnp.where(mask, t1v, 0.0))
                load_acc[e] += jnp.sum(jnp.where(mask, 1.0 / B, 0.0))
        lane = jax.lax.iota(jnp.int32, LANES)
        impv = zero16
        loadv = zero16
        for e in range(E):
            impv = jnp.where(lane == e, imp_acc[e], impv)
            loadv = jnp.where(lane == e, load_acc[e], loadv)
        imp_v[...] = impv
        load_v[...] = loadv
        pltpu.async_copy(ew_v, ew_hbm, sem).wait()
        pltpu.async_copy(imp_v, imp_hbm, sem).wait()
        pltpu.async_copy(load_v, load_hbm, sem).wait()


@jax.jit
def kernel(tokens, spatial_shape, gate_W, gate_b, W_down, b_down, W_up, b_up):
    del spatial_shape
    out, logits, sel, selt, t1t = pl.pallas_call(
        _moe_kernel,
        grid=(RSTEPS + ASTEPS,),
        in_specs=[
            pl.BlockSpec((RB, T, D),
                         lambda i: (jnp.minimum(i, RSTEPS - 1), 0, 0)),
            pl.BlockSpec((SB, T, D),
                         lambda i: (jnp.maximum(i - RSTEPS, 0), 0, 0)),
            pl.BlockSpec((D, E), lambda i: (0, 0)),
            pl.BlockSpec((1, E), lambda i: (0, 0)),
            pl.BlockSpec((E, D, R), lambda i: (0, 0, 0)),
            pl.BlockSpec((E, R, D), lambda i: (0, 0, 0)),
            pl.BlockSpec((E, R), lambda i: (0, 0)),
            pl.BlockSpec((E, D), lambda i: (0, 0)),
        ],
        out_specs=[
            pl.BlockSpec((SB, T, D),
                         lambda i: (jnp.maximum(i - RSTEPS, 0), 0, 0)),
            pl.BlockSpec((B, E), lambda i: (0, 0)),
            pl.BlockSpec((B, 1), lambda i: (0, 0)),
            pl.BlockSpec((1, B), lambda i: (0, 0)),
            pl.BlockSpec((1, B), lambda i: (0, 0)),
        ],
        out_shape=[
            jax.ShapeDtypeStruct((B, T, D), jnp.float32),
            jax.ShapeDtypeStruct((B, E), jnp.float32),
            jax.ShapeDtypeStruct((B, 1), jnp.int32),
            jax.ShapeDtypeStruct((1, B), jnp.int32),
            jax.ShapeDtypeStruct((1, B), jnp.float32),
        ],
        scratch_shapes=[
            pltpu.VMEM((RSTEPS, RB, D), jnp.float32),
            pltpu.SMEM((1, B), jnp.int32),
            pltpu.SMEM((1, B), jnp.float32),
            pltpu.SemaphoreType.DMA,
        ],
        compiler_params=pltpu.CompilerParams(
            dimension_semantics=("arbitrary",)),
    )(tokens, tokens, gate_W, gate_b.reshape(1, E),
      W_down.astype(jnp.bfloat16), W_up.astype(jnp.bfloat16), b_down, b_up)

    sc_gating = pl.kernel(
        _sc_gating_kernel,
        out_type=[
            jax.ShapeDtypeStruct((B * E,), jnp.float32),
            jax.ShapeDtypeStruct((LANES,), jnp.float32),
            jax.ShapeDtypeStruct((LANES,), jnp.float32),
        ],
        mesh=plsc.VectorSubcoreMesh(core_axis_name="c", subcore_axis_name="s"),
        compiler_params=_SC_COMPILER_PARAMS,
        scratch_types=[
            pltpu.VMEM((1, B), jnp.int32),
            pltpu.VMEM((1, B), jnp.float32),
            pltpu.VMEM((B * E,), jnp.float32),
            pltpu.VMEM((LANES,), jnp.float32),
            pltpu.VMEM((LANES,), jnp.float32),
            pltpu.SemaphoreType.DMA,
        ],
    )
    ew_flat, imp16, load16 = sc_gating(selt, t1t)

    return (out, logits, sel, ew_flat.reshape(B, E), imp16[:E], load16[:E])
